# baseline (speedup 1.0000x reference)
import functools
import math

import jax
import jax.numpy as jnp
from jax import lax
from jax.experimental import pallas as pl
from jax.experimental.pallas import tpu as pltpu
from jax.experimental.pallas import tpu_sc as plsc

VOCAB = 100000
EMB = 64
HID = 128
BATCH = 1024

TV = 1024
NT = math.ceil(VOCAB / TV)

_NC, _NS = 2, 16
_NW = _NC * _NS
_BPW = BATCH // _NW

@functools.cache
def _get_sc_gather():
    mesh = plsc.VectorSubcoreMesh(core_axis_name="c", subcore_axis_name="s")

    @functools.partial(
        pl.kernel,
        mesh=mesh,
        out_type=jax.ShapeDtypeStruct((BATCH, EMB), jnp.float32),
        scratch_types=[
            pltpu.VMEM((_BPW,), jnp.int32),
            pltpu.VMEM((_BPW, EMB), jnp.float32),
            pltpu.SemaphoreType.DMA,
        ],
        compiler_params=pltpu.CompilerParams(use_tc_tiling_on_sc=False),
    )
    def sc_gather(table_hbm, idx_hbm, out_hbm, idx_v, rows_v, sem):
        wid = lax.axis_index("s") * _NC + lax.axis_index("c")
        base = wid * _BPW
        pltpu.sync_copy(idx_hbm.at[pl.ds(base, _BPW)], idx_v)
        pltpu.async_copy(table_hbm.at[idx_v], rows_v, sem).wait()
        pltpu.sync_copy(rows_v, out_hbm.at[pl.ds(base, _BPW)])

    return sc_gather


def _stats_body(emb_ref, w1_ref, b1_ref, w2_ref, b2_ref,
                h_ref, lse_ref, m_ref, s_ref):
    t = pl.program_id(0)

    @pl.when(t == 0)
    def _init():
        h = jnp.dot(emb_ref[...], w1_ref[...],
                    preferred_element_type=jnp.float32) + b1_ref[...]
        h_ref[...] = jnp.maximum(h, 0.0)
        m_ref[...] = jnp.full((BATCH, 1), -jnp.inf, jnp.float32)
        s_ref[...] = jnp.zeros((BATCH, 1), jnp.float32)

    logits = jnp.dot(h_ref[...], w2_ref[...],
                     preferred_element_type=jnp.float32) + b2_ref[...]
    col = t * TV + lax.broadcasted_iota(jnp.int32, (1, TV), 1)
    logits = jnp.where(col < VOCAB, logits, -jnp.inf)
    m_old = m_ref[...]
    m_new = jnp.maximum(m_old, jnp.max(logits, axis=1, keepdims=True))
    s_ref[...] = (s_ref[...] * jnp.exp(m_old - m_new)
                  + jnp.sum(jnp.exp(logits - m_new), axis=1, keepdims=True))
    m_ref[...] = m_new

    @pl.when(t == NT - 1)
    def _fin():
        lse_ref[...] = m_new + jnp.log(s_ref[...])


def _write_body(h_ref, lse_ref, w2_ref, b2_ref, out_ref):
    logits = jnp.dot(h_ref[...], w2_ref[...],
                     preferred_element_type=jnp.float32) + b2_ref[...]
    out_ref[...] = logits - lse_ref[...]


def kernel(inputs, emb_table, W1, b1, W2, b2):
    b1r = b1.reshape(1, HID)
    b2r = b2.reshape(1, VOCAB)

    embeds = _get_sc_gather()(emb_table, inputs)

    h, lse = pl.pallas_call(
        _stats_body,
        grid=(NT,),
        in_specs=[
            pl.BlockSpec((BATCH, EMB), lambda t: (0, 0)),
            pl.BlockSpec((EMB, HID), lambda t: (0, 0)),
            pl.BlockSpec((1, HID), lambda t: (0, 0)),
            pl.BlockSpec((HID, TV), lambda t: (0, t)),
            pl.BlockSpec((1, TV), lambda t: (0, t)),
        ],
        out_specs=[
            pl.BlockSpec((BATCH, HID), lambda t: (0, 0)),
            pl.BlockSpec((BATCH, 1), lambda t: (0, 0)),
        ],
        out_shape=[
            jax.ShapeDtypeStruct((BATCH, HID), jnp.float32),
            jax.ShapeDtypeStruct((BATCH, 1), jnp.float32),
        ],
        scratch_shapes=[
            pltpu.VMEM((BATCH, 1), jnp.float32),
            pltpu.VMEM((BATCH, 1), jnp.float32),
        ],
    )(embeds, W1, b1r, W2, b2r)

    log_probs = pl.pallas_call(
        _write_body,
        grid=(NT,),
        in_specs=[
            pl.BlockSpec((BATCH, HID), lambda t: (0, 0)),
            pl.BlockSpec((BATCH, 1), lambda t: (0, 0)),
            pl.BlockSpec((HID, TV), lambda t: (0, t)),
            pl.BlockSpec((1, TV), lambda t: (0, t)),
        ],
        out_specs=pl.BlockSpec((BATCH, TV), lambda t: (0, t)),
        out_shape=jax.ShapeDtypeStruct((BATCH, VOCAB), jnp.float32),
        compiler_params=pltpu.CompilerParams(
            dimension_semantics=("arbitrary",),
        ),
    )(h, lse, W2, b2r)

    return log_probs


# speedup vs baseline: 1.0071x; 1.0071x over previous
import functools
import math

import jax
import jax.numpy as jnp
from jax import lax
from jax.experimental import pallas as pl
from jax.experimental.pallas import tpu as pltpu
from jax.experimental.pallas import tpu_sc as plsc

VOCAB = 100000
EMB = 64
HID = 128
BATCH = 1024

TV = 1024
NT = math.ceil(VOCAB / TV)

_NC, _NS = 2, 16
_NW = _NC * _NS
_BPW = BATCH // _NW

@functools.cache
def _get_sc_gather():
    mesh = plsc.VectorSubcoreMesh(core_axis_name="c", subcore_axis_name="s")

    @functools.partial(
        pl.kernel,
        mesh=mesh,
        out_type=jax.ShapeDtypeStruct((BATCH, EMB), jnp.float32),
        scratch_types=[
            pltpu.VMEM((_BPW,), jnp.int32),
            pltpu.VMEM((_BPW, EMB), jnp.float32),
            pltpu.SemaphoreType.DMA,
        ],
        compiler_params=pltpu.CompilerParams(use_tc_tiling_on_sc=False),
    )
    def sc_gather(table_hbm, idx_hbm, out_hbm, idx_v, rows_v, sem):
        wid = lax.axis_index("s") * _NC + lax.axis_index("c")
        base = wid * _BPW
        pltpu.sync_copy(idx_hbm.at[pl.ds(base, _BPW)], idx_v)
        pltpu.async_copy(table_hbm.at[idx_v], rows_v, sem).wait()
        pltpu.sync_copy(rows_v, out_hbm.at[pl.ds(base, _BPW)])

    return sc_gather


def _stats_body(emb_ref, w1_ref, b1_ref, w2_ref, b2_ref,
                h_ref, lse_ref, m_ref, s_ref):
    t = pl.program_id(0)

    @pl.when(t == 0)
    def _init():
        h = jnp.dot(emb_ref[...], w1_ref[...],
                    preferred_element_type=jnp.float32) + b1_ref[...]
        h_ref[...] = jnp.maximum(h, 0.0)
        m_ref[...] = jnp.full((BATCH, 1), -jnp.inf, jnp.float32)
        s_ref[...] = jnp.zeros((BATCH, 1), jnp.float32)

    logits = jnp.dot(h_ref[...].astype(jnp.bfloat16),
                     w2_ref[...].astype(jnp.bfloat16),
                     preferred_element_type=jnp.float32) + b2_ref[...]
    col = t * TV + lax.broadcasted_iota(jnp.int32, (1, TV), 1)
    logits = jnp.where(col < VOCAB, logits, -jnp.inf)
    m_old = m_ref[...]
    m_new = jnp.maximum(m_old, jnp.max(logits, axis=1, keepdims=True))
    s_ref[...] = (s_ref[...] * jnp.exp(m_old - m_new)
                  + jnp.sum(jnp.exp(logits - m_new), axis=1, keepdims=True))
    m_ref[...] = m_new

    @pl.when(t == NT - 1)
    def _fin():
        lse_ref[...] = m_new + jnp.log(s_ref[...])


def _write_body(h_ref, lse_ref, w2_ref, b2_ref, out_ref):
    logits = jnp.dot(h_ref[...].astype(jnp.bfloat16),
                     w2_ref[...].astype(jnp.bfloat16),
                     preferred_element_type=jnp.float32) + b2_ref[...]
    out_ref[...] = logits - lse_ref[...]


def kernel(inputs, emb_table, W1, b1, W2, b2):
    b1r = b1.reshape(1, HID)
    b2r = b2.reshape(1, VOCAB)

    embeds = _get_sc_gather()(emb_table, inputs)

    h, lse = pl.pallas_call(
        _stats_body,
        grid=(NT,),
        in_specs=[
            pl.BlockSpec((BATCH, EMB), lambda t: (0, 0)),
            pl.BlockSpec((EMB, HID), lambda t: (0, 0)),
            pl.BlockSpec((1, HID), lambda t: (0, 0)),
            pl.BlockSpec((HID, TV), lambda t: (0, t)),
            pl.BlockSpec((1, TV), lambda t: (0, t)),
        ],
        out_specs=[
            pl.BlockSpec((BATCH, HID), lambda t: (0, 0)),
            pl.BlockSpec((BATCH, 1), lambda t: (0, 0)),
        ],
        out_shape=[
            jax.ShapeDtypeStruct((BATCH, HID), jnp.float32),
            jax.ShapeDtypeStruct((BATCH, 1), jnp.float32),
        ],
        scratch_shapes=[
            pltpu.VMEM((BATCH, 1), jnp.float32),
            pltpu.VMEM((BATCH, 1), jnp.float32),
        ],
    )(embeds, W1, b1r, W2, b2r)

    log_probs = pl.pallas_call(
        _write_body,
        grid=(NT,),
        in_specs=[
            pl.BlockSpec((BATCH, HID), lambda t: (0, 0)),
            pl.BlockSpec((BATCH, 1), lambda t: (0, 0)),
            pl.BlockSpec((HID, TV), lambda t: (0, t)),
            pl.BlockSpec((1, TV), lambda t: (0, t)),
        ],
        out_specs=pl.BlockSpec((BATCH, TV), lambda t: (0, t)),
        out_shape=jax.ShapeDtypeStruct((BATCH, VOCAB), jnp.float32),
        compiler_params=pltpu.CompilerParams(
            dimension_semantics=("arbitrary",),
        ),
    )(h, lse, W2, b2r)

    return log_probs


# speedup vs baseline: 2.0910x; 2.0762x over previous
import functools
import math

import jax
import jax.numpy as jnp
from jax import lax
from jax.experimental import pallas as pl
from jax.experimental.pallas import tpu as pltpu
from jax.experimental.pallas import tpu_sc as plsc

VOCAB = 100000
EMB = 64
HID = 128
BATCH = 1024

TV = 1024
NT = math.ceil(VOCAB / TV)

_NC, _NS = 2, 16
_NW = _NC * _NS
_BPW = BATCH // _NW


@functools.cache
def _get_sc_gather():
    mesh = plsc.VectorSubcoreMesh(core_axis_name="c", subcore_axis_name="s")

    @functools.partial(
        pl.kernel,
        mesh=mesh,
        out_type=jax.ShapeDtypeStruct((BATCH, EMB), jnp.float32),
        scratch_types=[
            pltpu.VMEM((_BPW,), jnp.int32),
            pltpu.VMEM((_BPW, EMB), jnp.float32),
            pltpu.SemaphoreType.DMA,
        ],
        compiler_params=pltpu.CompilerParams(use_tc_tiling_on_sc=False),
    )
    def sc_gather(table_hbm, idx_hbm, out_hbm, idx_v, rows_v, sem):
        wid = lax.axis_index("s") * _NC + lax.axis_index("c")
        base = wid * _BPW
        pltpu.sync_copy(idx_hbm.at[pl.ds(base, _BPW)], idx_v)
        pltpu.async_copy(table_hbm.at[idx_v], rows_v, sem).wait()
        pltpu.sync_copy(rows_v, out_hbm.at[pl.ds(base, _BPW)])

    return sc_gather


def _stats_body(emb_ref, w1_ref, b1_ref, w2t_ref, b2_ref,
                ht_ref, lse_ref, m_ref, s_ref):
    t = pl.program_id(0)

    @pl.when(t == 0)
    def _init():
        h = jnp.dot(emb_ref[...], w1_ref[...],
                    preferred_element_type=jnp.float32) + b1_ref[...]
        ht_ref[...] = jnp.maximum(h, 0.0).T
        m_ref[...] = jnp.full((1, BATCH), -jnp.inf, jnp.float32)
        s_ref[...] = jnp.zeros((1, BATCH), jnp.float32)

    logits = jnp.dot(w2t_ref[...].astype(jnp.bfloat16),
                     ht_ref[...].astype(jnp.bfloat16),
                     preferred_element_type=jnp.float32) + b2_ref[...].T
    row = t * TV + lax.broadcasted_iota(jnp.int32, (TV, 1), 0)
    logits = jnp.where(row < VOCAB, logits, -jnp.inf)
    m_old = m_ref[...]
    m_new = jnp.maximum(m_old, jnp.max(logits, axis=0, keepdims=True))
    s_ref[...] = (s_ref[...] * jnp.exp(m_old - m_new)
                  + jnp.sum(jnp.exp(logits - m_new), axis=0, keepdims=True))
    m_ref[...] = m_new

    @pl.when(t == NT - 1)
    def _fin():
        lse_ref[...] = m_new + jnp.log(s_ref[...])


def _write_body(ht_ref, lse_ref, w2t_ref, b2_ref, out_ref):
    logits = jnp.dot(w2t_ref[...].astype(jnp.bfloat16),
                     ht_ref[...].astype(jnp.bfloat16),
                     preferred_element_type=jnp.float32) + b2_ref[...].T
    out_ref[...] = logits - lse_ref[...]


def kernel(inputs, emb_table, W1, b1, W2, b2):
    b1r = b1.reshape(1, HID)
    b2r = b2.reshape(1, VOCAB)
    W2T = W2.T

    embeds = _get_sc_gather()(emb_table, inputs)

    ht, lse = pl.pallas_call(
        _stats_body,
        grid=(NT,),
        in_specs=[
            pl.BlockSpec((BATCH, EMB), lambda t: (0, 0)),
            pl.BlockSpec((EMB, HID), lambda t: (0, 0)),
            pl.BlockSpec((1, HID), lambda t: (0, 0)),
            pl.BlockSpec((TV, HID), lambda t: (t, 0)),
            pl.BlockSpec((1, TV), lambda t: (0, t)),
        ],
        out_specs=[
            pl.BlockSpec((HID, BATCH), lambda t: (0, 0)),
            pl.BlockSpec((1, BATCH), lambda t: (0, 0)),
        ],
        out_shape=[
            jax.ShapeDtypeStruct((HID, BATCH), jnp.float32),
            jax.ShapeDtypeStruct((1, BATCH), jnp.float32),
        ],
        scratch_shapes=[
            pltpu.VMEM((1, BATCH), jnp.float32),
            pltpu.VMEM((1, BATCH), jnp.float32),
        ],
    )(embeds, W1, b1r, W2T, b2r)

    log_probs_t = pl.pallas_call(
        _write_body,
        grid=(NT,),
        in_specs=[
            pl.BlockSpec((HID, BATCH), lambda t: (0, 0)),
            pl.BlockSpec((1, BATCH), lambda t: (0, 0)),
            pl.BlockSpec((TV, HID), lambda t: (t, 0)),
            pl.BlockSpec((1, TV), lambda t: (0, t)),
        ],
        out_specs=pl.BlockSpec((TV, BATCH), lambda t: (t, 0)),
        out_shape=jax.ShapeDtypeStruct((VOCAB, BATCH), jnp.float32),
        compiler_params=pltpu.CompilerParams(
            dimension_semantics=("arbitrary",),
        ),
    )(ht, lse, W2T, b2r)

    return log_probs_t.T
